# baseline (speedup 1.0000x reference)
import jax
import jax.numpy as jnp
from jax.experimental import pallas as pl
from jax.experimental.pallas import tpu as pltpu

N = 10000
F = 128

BM = 200
NB = N // BM


def _resgcn_kernel(x_ref, adj_ref, w_ref, b_ref, w1_ref, b1_ref, w2_ref,
                   b2_ref, out_ref, z_s, s1_s, s2_s, x1_s):
    t = pl.program_id(0)

    @pl.when(t == 0)
    def _prologue():
        xv = x_ref[...]
        z_s[...] = (
            jnp.dot(xv, w_ref[...], preferred_element_type=jnp.float32)
            + b_ref[...]
        )
        s1_s[...] = jnp.dot(xv, w1_ref[...], preferred_element_type=jnp.float32)

    @pl.when(t == NB)
    def _mid():
        s2_s[...] = jnp.dot(
            x1_s[...], w2_ref[...], preferred_element_type=jnp.float32
        )

    a = adj_ref[...]

    @pl.when(t < NB)
    def _gc1():
        g = jnp.dot(a, s1_s[...], preferred_element_type=jnp.float32)
        row = pl.ds(t * BM, BM)
        x1_s[row, :] = jnp.maximum(g + b1_ref[...], 0.0) + z_s[row, :]

    @pl.when(t >= NB)
    def _gc2():
        g = jnp.dot(a, s2_s[...], preferred_element_type=jnp.float32)
        g = g + b2_ref[...]
        m = jnp.max(g, axis=1, keepdims=True)
        shifted = g - m
        lse = jnp.log(jnp.sum(jnp.exp(shifted), axis=1, keepdims=True))
        out_ref[...] = shifted - lse


@jax.jit
def _run(x, adj, W, b, W1, b1, W2, b2):
    full = pl.BlockSpec((N, F), lambda t: (0, 0))
    wspec = pl.BlockSpec((F, F), lambda t: (0, 0))
    bspec = pl.BlockSpec((1, F), lambda t: (0, 0))
    adj_spec = pl.BlockSpec((BM, N), lambda t: (jax.lax.rem(t, NB), 0))
    out_spec = pl.BlockSpec(
        (BM, F), lambda t: (jnp.maximum(t - NB, 0), 0)
    )

    return pl.pallas_call(
        _resgcn_kernel,
        grid=(2 * NB,),
        in_specs=[full, adj_spec, wspec, bspec, wspec, bspec, wspec, bspec],
        out_specs=out_spec,
        out_shape=jax.ShapeDtypeStruct((N, F), jnp.float32),
        scratch_shapes=[
            pltpu.VMEM((N, F), jnp.float32),
            pltpu.VMEM((N, F), jnp.float32),
            pltpu.VMEM((N, F), jnp.float32),
            pltpu.VMEM((N, F), jnp.float32),
        ],
        compiler_params=pltpu.CompilerParams(
            dimension_semantics=("arbitrary",),
        ),
    )(x, adj, W, b.reshape(1, F), W1, b1.reshape(1, F), W2, b2.reshape(1, F))


def kernel(x, adj, W, b, W1, b1, W2, b2):
    return _run(x, adj, W, b, W1, b1, W2, b2)


# speedup vs baseline: 1.0481x; 1.0481x over previous
import jax
import jax.numpy as jnp
from jax.experimental import pallas as pl
from jax.experimental.pallas import tpu as pltpu

N = 10000
F = 128

BM = 400


def _pre_kernel(x_ref, w1_ref, s1_ref):
    s1_ref[...] = jnp.dot(
        x_ref[...], w1_ref[...], preferred_element_type=jnp.float32
    )


def _gc1_kernel(adj_ref, s1_ref, x_ref, w_ref, b_ref, b1_ref, w2_ref, s2_ref):
    g = jnp.dot(adj_ref[...], s1_ref[...], preferred_element_type=jnp.float32)
    z = (
        jnp.dot(x_ref[...], w_ref[...], preferred_element_type=jnp.float32)
        + b_ref[...]
    )
    x1 = jnp.maximum(g + b1_ref[...], 0.0) + z
    s2_ref[...] = jnp.dot(x1, w2_ref[...], preferred_element_type=jnp.float32)


def _gc2_kernel(adj_ref, s2_ref, b2_ref, out_ref):
    g = jnp.dot(adj_ref[...], s2_ref[...], preferred_element_type=jnp.float32)
    g = g + b2_ref[...]
    m = jnp.max(g, axis=1, keepdims=True)
    shifted = g - m
    lse = jnp.log(jnp.sum(jnp.exp(shifted), axis=1, keepdims=True))
    out_ref[...] = shifted - lse


@jax.jit
def _run(x, adj, W, b, W1, b1, W2, b2):
    s1 = pl.pallas_call(
        _pre_kernel,
        out_shape=jax.ShapeDtypeStruct((N, F), jnp.float32),
    )(x, W1)

    grid = (N // BM,)
    row_spec = pl.BlockSpec((BM, F), lambda i: (i, 0))
    full_spec = pl.BlockSpec((N, F), lambda i: (0, 0))
    bias_spec = pl.BlockSpec((1, F), lambda i: (0, 0))
    w_spec = pl.BlockSpec((F, F), lambda i: (0, 0))
    adj_spec = pl.BlockSpec((BM, N), lambda i: (i, 0))
    params = pltpu.CompilerParams(dimension_semantics=("parallel",))

    s2 = pl.pallas_call(
        _gc1_kernel,
        grid=grid,
        in_specs=[adj_spec, full_spec, row_spec, w_spec, bias_spec, bias_spec,
                  w_spec],
        out_specs=row_spec,
        out_shape=jax.ShapeDtypeStruct((N, F), jnp.float32),
        compiler_params=params,
    )(adj, s1, x, W, b.reshape(1, F), b1.reshape(1, F), W2)

    out = pl.pallas_call(
        _gc2_kernel,
        grid=grid,
        in_specs=[adj_spec, full_spec, bias_spec],
        out_specs=row_spec,
        out_shape=jax.ShapeDtypeStruct((N, F), jnp.float32),
        compiler_params=params,
    )(adj, s2, b2.reshape(1, F))

    return out


def kernel(x, adj, W, b, W1, b1, W2, b2):
    return _run(x, adj, W, b, W1, b1, W2, b2)


# speedup vs baseline: 1.0694x; 1.0203x over previous
import jax
import jax.numpy as jnp
from jax.experimental import pallas as pl
from jax.experimental.pallas import tpu as pltpu

N = 10000
F = 128

BM = 400


def _gc1_kernel(adj_ref, xfull_ref, x_ref, w_ref, b_ref, w1_ref, b1_ref,
                w2_ref, s2_ref):
    h = jnp.dot(adj_ref[...], xfull_ref[...], preferred_element_type=jnp.float32)
    g = jnp.dot(h, w1_ref[...], preferred_element_type=jnp.float32)
    z = (
        jnp.dot(x_ref[...], w_ref[...], preferred_element_type=jnp.float32)
        + b_ref[...]
    )
    x1 = jnp.maximum(g + b1_ref[...], 0.0) + z
    s2_ref[...] = jnp.dot(x1, w2_ref[...], preferred_element_type=jnp.float32)


def _gc2_kernel(adj_ref, s2_ref, b2_ref, out_ref):
    g = jnp.dot(adj_ref[...], s2_ref[...], preferred_element_type=jnp.float32)
    g = g + b2_ref[...]
    m = jnp.max(g, axis=1, keepdims=True)
    shifted = g - m
    lse = jnp.log(jnp.sum(jnp.exp(shifted), axis=1, keepdims=True))
    out_ref[...] = shifted - lse


@jax.jit
def _run(x, adj, W, b, W1, b1, W2, b2):
    grid = (N // BM,)
    row_spec = pl.BlockSpec((BM, F), lambda i: (i, 0))
    full_spec = pl.BlockSpec((N, F), lambda i: (0, 0))
    bias_spec = pl.BlockSpec((1, F), lambda i: (0, 0))
    w_spec = pl.BlockSpec((F, F), lambda i: (0, 0))
    adj_spec = pl.BlockSpec((BM, N), lambda i: (i, 0))
    params = pltpu.CompilerParams(dimension_semantics=("parallel",))

    s2 = pl.pallas_call(
        _gc1_kernel,
        grid=grid,
        in_specs=[adj_spec, full_spec, row_spec, w_spec, bias_spec, w_spec,
                  bias_spec, w_spec],
        out_specs=row_spec,
        out_shape=jax.ShapeDtypeStruct((N, F), jnp.float32),
        compiler_params=params,
    )(adj, x, x, W, b.reshape(1, F), W1, b1.reshape(1, F), W2)

    out = pl.pallas_call(
        _gc2_kernel,
        grid=grid,
        in_specs=[adj_spec, full_spec, bias_spec],
        out_specs=row_spec,
        out_shape=jax.ShapeDtypeStruct((N, F), jnp.float32),
        compiler_params=params,
    )(adj, s2, b2.reshape(1, F))

    return out


def kernel(x, adj, W, b, W1, b1, W2, b2):
    return _run(x, adj, W, b, W1, b1, W2, b2)


# speedup vs baseline: 1.0925x; 1.0216x over previous
import jax
import jax.numpy as jnp
from jax.experimental import pallas as pl
from jax.experimental.pallas import tpu as pltpu

N = 10000
F = 128

BM = 400


def _gc1_kernel(adj_ref, xfull_ref, w_ref, b_ref, w1_ref, b1_ref,
                w2_ref, s2_ref):
    i = pl.program_id(0)
    h = jnp.dot(adj_ref[...], xfull_ref[...], preferred_element_type=jnp.float32)
    g = jnp.dot(h, w1_ref[...], preferred_element_type=jnp.float32)
    xi = xfull_ref[pl.ds(i * BM, BM), :]
    z = (
        jnp.dot(xi, w_ref[...], preferred_element_type=jnp.float32)
        + b_ref[...]
    )
    x1 = jnp.maximum(g + b1_ref[...], 0.0) + z
    s2_ref[...] = jnp.dot(x1, w2_ref[...], preferred_element_type=jnp.float32)


def _gc2_kernel(adj_ref, s2_ref, b2_ref, out_ref):
    g = jnp.dot(adj_ref[...], s2_ref[...], preferred_element_type=jnp.float32)
    g = g + b2_ref[...]
    m = jnp.max(g, axis=1, keepdims=True)
    shifted = g - m
    lse = jnp.log(jnp.sum(jnp.exp(shifted), axis=1, keepdims=True))
    out_ref[...] = shifted - lse


@jax.jit
def _run(x, adj, W, b, W1, b1, W2, b2):
    grid = (N // BM,)
    row_spec = pl.BlockSpec((BM, F), lambda i: (i, 0))
    full_spec = pl.BlockSpec((N, F), lambda i: (0, 0))
    bias_spec = pl.BlockSpec((1, F), lambda i: (0, 0))
    w_spec = pl.BlockSpec((F, F), lambda i: (0, 0))
    adj_spec = pl.BlockSpec((BM, N), lambda i: (i, 0))
    params = pltpu.CompilerParams(dimension_semantics=("parallel",))

    s2 = pl.pallas_call(
        _gc1_kernel,
        grid=grid,
        in_specs=[adj_spec, full_spec, w_spec, bias_spec, w_spec,
                  bias_spec, w_spec],
        out_specs=row_spec,
        out_shape=jax.ShapeDtypeStruct((N, F), jnp.float32),
        compiler_params=params,
    )(adj, x, W, b.reshape(1, F), W1, b1.reshape(1, F), W2)

    out = pl.pallas_call(
        _gc2_kernel,
        grid=grid,
        in_specs=[adj_spec, full_spec, bias_spec],
        out_specs=row_spec,
        out_shape=jax.ShapeDtypeStruct((N, F), jnp.float32),
        compiler_params=params,
    )(adj, s2, b2.reshape(1, F))

    return out


def kernel(x, adj, W, b, W1, b1, W2, b2):
    return _run(x, adj, W, b, W1, b1, W2, b2)
